# baseline (speedup 1.0000x reference)
import jax
import jax.numpy as jnp
from jax.experimental import pallas as pl
from jax.experimental.pallas import tpu as pltpu

_ALPHA = 0.2
_ROW_BLOCK = 128


def _gat_kernel(nf_ref, W_ref, ws_ref, wdr_ref, Hw_ref, Hb_ref, b_ref, adj_ref,
                attn_ref, feat_ref, h_s, src_s, dstr_s, hx_s):
    H = W_ref.shape[0]
    O = W_ref.shape[2]
    bn = adj_ref.shape[1]
    r = pl.program_id(1)

    @pl.when(r == 0)
    def _init():
        nf = nf_ref[0]
        hx_s[...] = (
            jnp.dot(nf, Hw_ref[...], preferred_element_type=jnp.float32)
            + Hb_ref[...]
        )
        ones_col = jnp.ones((nf.shape[0], 1), jnp.float32)
        zpad = jnp.zeros((nf.shape[0], h_s.shape[2] - O - 1), jnp.float32)
        for h in range(H):
            hm = jnp.dot(nf, W_ref[h], preferred_element_type=jnp.float32)
            h_s[h] = jnp.concatenate([hm, ones_col, zpad], axis=1)
            t = jnp.tanh(hm)
            src_s[h] = jnp.minimum(
                jnp.dot(t, ws_ref[h], preferred_element_type=jnp.float32), 43.0)
            dstr_s[h] = jnp.minimum(jax.lax.dot_general(
                wdr_ref[h], t, (((1,), (1,)), ((), ())),
                preferred_element_type=jnp.float32), 43.0)

    adjb = adj_ref[0]
    feats = []
    for h in range(H):
        src = src_s[h, pl.ds(r * bn, bn), :]
        a = src + dstr_s[h]
        l = jnp.maximum(a, _ALPHA * a)
        e = jnp.exp2(l) * adjb
        q = jnp.dot(e, h_s[h], preferred_element_type=jnp.float32)
        recip = 1.0 / q[:, O:O + 1]
        attn_ref[0, h] = e * recip
        feats.append(q[:, :O] * recip)

    f = jnp.concatenate(feats, axis=1) + b_ref[...]
    hx = hx_s[pl.ds(r * bn, bn), :]
    f = jnp.where(f > 0, f, jnp.exp(jnp.minimum(f, 0.0)) - 1.0)
    g = jax.nn.sigmoid(hx)
    feat_ref[0] = g * f + (1.0 - g) * hx


def kernel(node_feature, adj, W, b, w_src, w_dst, Hw, Hb):
    B, N, D = node_feature.shape
    H, _, O = W.shape
    f32 = jnp.float32

    log2e = jnp.float32(1.4426950408889634)
    w_src_l2 = w_src * log2e
    w_dst_row = w_dst.transpose(0, 2, 1) * log2e
    Hb_row = Hb.reshape(1, H * O)
    b_row = jnp.tile(b, H).reshape(1, H * O)

    bn = _ROW_BLOCK
    nb = N // bn
    attn, feat = pl.pallas_call(
        _gat_kernel,
        grid=(B, nb),
        in_specs=[
            pl.BlockSpec((1, N, D), lambda bi, r: (bi, 0, 0)),
            pl.BlockSpec((H, D, O), lambda bi, r: (0, 0, 0)),
            pl.BlockSpec((H, O, 1), lambda bi, r: (0, 0, 0)),
            pl.BlockSpec((H, 1, O), lambda bi, r: (0, 0, 0)),
            pl.BlockSpec((D, H * O), lambda bi, r: (0, 0)),
            pl.BlockSpec((1, H * O), lambda bi, r: (0, 0)),
            pl.BlockSpec((1, H * O), lambda bi, r: (0, 0)),
            pl.BlockSpec((1, bn, N), lambda bi, r: (bi, r, 0)),
        ],
        out_specs=[
            pl.BlockSpec((1, H, bn, N), lambda bi, r: (bi, 0, r, 0)),
            pl.BlockSpec((1, bn, H * O), lambda bi, r: (bi, r, 0)),
        ],
        out_shape=[
            jax.ShapeDtypeStruct((B, H, N, N), f32),
            jax.ShapeDtypeStruct((B, N, H * O), f32),
        ],
        scratch_shapes=[
            pltpu.VMEM((H, N, 64), f32),
            pltpu.VMEM((H, N, 1), f32),
            pltpu.VMEM((H, 1, N), f32),
            pltpu.VMEM((N, H * O), f32),
        ],
        compiler_params=pltpu.CompilerParams(
            dimension_semantics=("arbitrary", "arbitrary"),
        ),
    )(node_feature, W, w_src_l2, w_dst_row, Hw, Hb_row, b_row, adj)

    return feat, attn


# speedup vs baseline: 1.1386x; 1.1386x over previous
import jax
import jax.numpy as jnp
from jax.experimental import pallas as pl
from jax.experimental.pallas import tpu as pltpu

_ALPHA = 0.2
_ROW_BLOCK = 256


def _gat_kernel(nf_ref, W_ref, ws_ref, wdr_ref, Hw_ref, Hb_ref, b_ref, adj_ref,
                attn_ref, feat_ref, h_s, src_s, dstr_s):
    H = W_ref.shape[0]
    O = W_ref.shape[2]
    bn = adj_ref.shape[1]
    r = pl.program_id(1)

    @pl.when(r == 0)
    def _init():
        nf = nf_ref[0]
        ones_col = jnp.ones((nf.shape[0], 1), jnp.float32)
        zpad = jnp.zeros((nf.shape[0], h_s.shape[2] - O - 1), jnp.float32)
        for h in range(H):
            hm = jnp.dot(nf, W_ref[h], preferred_element_type=jnp.float32)
            h_s[h] = jnp.concatenate([hm, ones_col, zpad], axis=1)
            t = jnp.tanh(hm)
            src_s[h] = jnp.minimum(
                jnp.dot(t, ws_ref[h], preferred_element_type=jnp.float32), 43.0)
            dstr_s[h] = jnp.minimum(jax.lax.dot_general(
                wdr_ref[h], t, (((1,), (1,)), ((), ())),
                preferred_element_type=jnp.float32), 43.0)

    adjb = adj_ref[0]
    feats = []
    for h in range(H):
        src = src_s[h, pl.ds(r * bn, bn), :]
        a = src + dstr_s[h]
        l = jnp.maximum(a, _ALPHA * a)
        e = jnp.exp2(l) * adjb
        q = jnp.dot(e, h_s[h], preferred_element_type=jnp.float32)
        recip = 1.0 / q[:, O:O + 1]
        attn_ref[0, h] = e * recip
        feats.append(q[:, :O] * recip)

    f = jnp.concatenate(feats, axis=1) + b_ref[...]
    hx = (
        jnp.dot(nf_ref[0, pl.ds(r * bn, bn), :], Hw_ref[...],
                preferred_element_type=jnp.float32)
        + Hb_ref[...]
    )
    f = jnp.where(f > 0, f, jnp.exp(jnp.minimum(f, 0.0)) - 1.0)
    g = jax.nn.sigmoid(hx)
    feat_ref[0] = g * f + (1.0 - g) * hx


def kernel(node_feature, adj, W, b, w_src, w_dst, Hw, Hb):
    B, N, D = node_feature.shape
    H, _, O = W.shape
    f32 = jnp.float32

    log2e = jnp.float32(1.4426950408889634)
    w_src_l2 = w_src * log2e
    w_dst_row = w_dst.transpose(0, 2, 1) * log2e
    Hb_row = Hb.reshape(1, H * O)
    b_row = jnp.tile(b, H).reshape(1, H * O)

    bn = _ROW_BLOCK
    nb = N // bn
    attn, feat = pl.pallas_call(
        _gat_kernel,
        grid=(B, nb),
        in_specs=[
            pl.BlockSpec((1, N, D), lambda bi, r: (bi, 0, 0)),
            pl.BlockSpec((H, D, O), lambda bi, r: (0, 0, 0)),
            pl.BlockSpec((H, O, 1), lambda bi, r: (0, 0, 0)),
            pl.BlockSpec((H, 1, O), lambda bi, r: (0, 0, 0)),
            pl.BlockSpec((D, H * O), lambda bi, r: (0, 0)),
            pl.BlockSpec((1, H * O), lambda bi, r: (0, 0)),
            pl.BlockSpec((1, H * O), lambda bi, r: (0, 0)),
            pl.BlockSpec((1, bn, N), lambda bi, r: (bi, r, 0)),
        ],
        out_specs=[
            pl.BlockSpec((1, H, bn, N), lambda bi, r: (bi, 0, r, 0)),
            pl.BlockSpec((1, bn, H * O), lambda bi, r: (bi, r, 0)),
        ],
        out_shape=[
            jax.ShapeDtypeStruct((B, H, N, N), f32),
            jax.ShapeDtypeStruct((B, N, H * O), f32),
        ],
        scratch_shapes=[
            pltpu.VMEM((H, N, 64), f32),
            pltpu.VMEM((H, N, 1), f32),
            pltpu.VMEM((H, 1, N), f32),
        ],
        compiler_params=pltpu.CompilerParams(
            dimension_semantics=("arbitrary", "arbitrary"),
        ),
    )(node_feature, W, w_src_l2, w_dst_row, Hw, Hb_row, b_row, adj)

    return feat, attn
